# baseline (speedup 1.0000x reference)
import jax
import jax.numpy as jnp
from jax.experimental import pallas as pl

N = 10000
K = 32


def _knn_graph(c, k):
    n = c.shape[0]
    chunk = 1000
    nbrs = []
    for s in range(0, n, chunk):
        ci = c[s:s + chunk]
        m = ci.shape[0]
        d2 = jnp.sum((ci[:, None, :] - c[None, :, :]) ** 2, axis=-1)
        rows = jnp.arange(m)
        d2 = d2.at[rows, s + rows].set(jnp.inf)
        _, idx = jax.lax.top_k(-d2, k)
        nbrs.append(idx)
    nbr = jnp.concatenate(nbrs, axis=0)
    src = nbr.reshape(-1).astype(jnp.int32)
    dst = jnp.repeat(jnp.arange(n, dtype=jnp.int32), k)
    return jnp.stack([src, dst], axis=0)


def _gcn_conv(xh, W, b, row, col, ew, n):
    loop = jnp.arange(n, dtype=row.dtype)
    r = jnp.concatenate([row, loop])
    cc = jnp.concatenate([col, loop])
    w = jnp.concatenate([ew, jnp.ones((n,), ew.dtype)])
    deg = jnp.zeros((n,), ew.dtype).at[cc].add(w)
    dinv = jnp.where(deg > 0, 1.0 / jnp.sqrt(deg), 0.0)
    norm = dinv[r] * w * dinv[cc]
    h = xh @ W
    out = jnp.zeros((n, h.shape[1]), h.dtype).at[cc].add(norm[:, None] * h[r])
    return out + b


def _copy_kernel(a_ref, o_ref):
    o_ref[...] = a_ref[...]


def kernel(c, x, fc_c_W, fc_c_b, fc_a_W, fc_a_b, g1_W, g1_b, g2_W, g2_b, g3_W, g3_b,
           f1_W, f1_b, f2_W, f2_b, f3_W, f3_b, d1_W, d1_b, d2_W, d2_b):
    n = c.shape[0]
    ei = _knn_graph(c, K)
    row, col = ei[0], ei[1]
    dist = jnp.linalg.norm(c[row] - c[col], axis=1)
    ew = jnp.exp(-dist / 2.0)
    ew = jnp.where(ew > 10.0, jnp.zeros((), ew.dtype), ew)
    coords0 = c @ fc_c_W + fc_c_b
    attri0 = x @ fc_a_W + fc_a_b
    g1 = jax.nn.relu(_gcn_conv(c, g1_W, g1_b, row, col, ew, n))
    m1 = jax.nn.relu(x @ f1_W + f1_b)
    g1 = coords0 + g1
    m1 = attri0 + m1
    g2 = jax.nn.relu(_gcn_conv(g1, g2_W, g2_b, row, col, ew, n))
    m2 = jax.nn.relu(m1 @ f2_W + f2_b)
    g2 = g1 + g2
    m2 = m1 + m2
    g3 = jax.nn.relu(_gcn_conv(g2, g3_W, g3_b, row, col, ew, n))
    m3 = jax.nn.relu(m2 @ f3_W + f3_b)
    g3 = g2 + g3
    m3 = m2 + m3
    comb = jnp.concatenate([g3, m3], axis=1)
    h = jax.nn.relu(comb @ d1_W + d1_b)
    out = h @ d2_W + d2_b
    mean = out[:, 0]
    std = out[:, 1]
    sigma = 0.2 + 0.8 * jax.nn.softplus(std)
    g3 = pl.pallas_call(
        _copy_kernel,
        out_shape=jax.ShapeDtypeStruct(g3.shape, g3.dtype),
    )(g3)
    return (mean.reshape(-1, 1), sigma.reshape(-1, 1), g3, m3)


# speedup vs baseline: 3.3872x; 3.3872x over previous
import functools

import jax
import jax.numpy as jnp
from jax import lax
from jax.experimental import pallas as pl
from jax.experimental.pallas import tpu as pltpu
from jax.experimental.pallas import tpu_sc as plsc

N = 10000
K = 32
NP = 10240
BKNN = 128
BD = 1024
NW = 32
RPW = NP // NW
CHROWS = 4
GATHER = CHROWS * K
CPW = RPW // CHROWS

_HIGH = jax.lax.Precision.HIGHEST


CCH = 1024
NCH = NP // CCH


def _knn_body(cb_ref, ct_ref, idx_ref, ew_ref, dinv_ref, d2_ref):
    cb = cb_ref[...]
    cx = cb[:, 0:1]
    cy = cb[:, 1:2]
    r = pl.program_id(0) * BKNN + lax.broadcasted_iota(jnp.int32, (BKNN, 1), 0)
    jj = lax.broadcasted_iota(jnp.int32, (BKNN, CCH), 1)

    def fill(cc, _):
        ctx = ct_ref[cc, 0:1, :]
        cty = ct_ref[cc, 1:2, :]
        dx = cx - ctx
        dy = cy - cty
        d2 = dx * dx + dy * dy
        col = jj + cc * CCH
        d2_ref[cc] = jnp.where(col == r, jnp.inf, d2)
        return 0

    lax.fori_loop(0, NCH, fill, 0, unroll=False)

    def select(k, carry):
        vals, idxs = carry

        def mpass(cc, m):
            return jnp.minimum(m, jnp.min(d2_ref[cc], axis=1, keepdims=True))

        m = lax.fori_loop(0, NCH, mpass,
                          jnp.full((BKNN, 1), jnp.inf, jnp.float32),
                          unroll=False)

        def ipass(cc, i1):
            col = jj + cc * CCH
            cand = jnp.min(jnp.where(d2_ref[cc] == m, col, NP),
                           axis=1, keepdims=True)
            return jnp.minimum(i1, cand)

        i1 = lax.fori_loop(0, NCH, ipass,
                           jnp.full((BKNN, 1), NP, jnp.int32), unroll=False)

        def mask(cc, _):
            col = jj + cc * CCH
            d2_ref[cc] = jnp.where(col == i1, jnp.inf, d2_ref[cc])
            return 0

        lax.fori_loop(0, NCH, mask, 0, unroll=False)

        kcol = lax.broadcasted_iota(jnp.int32, (BKNN, K), 1)
        vals = jnp.where(kcol == k, m, vals)
        idxs = jnp.where(kcol == k, i1, idxs)
        return vals, idxs

    vals0 = jnp.zeros((BKNN, K), jnp.float32)
    idxs0 = jnp.zeros((BKNN, K), jnp.int32)
    v, ii = lax.fori_loop(0, K, select, (vals0, idxs0), unroll=False)
    ew = jnp.exp(-jnp.sqrt(v) / 2.0)
    idx_ref[...] = ii
    ew_ref[...] = ew
    dinv_ref[...] = 1.0 / jnp.sqrt(1.0 + jnp.sum(ew, axis=1, keepdims=True))


def _knn(c_blocks, c_pad3):
    grid = NP // BKNN
    return pl.pallas_call(
        _knn_body,
        grid=(grid,),
        in_specs=[
            pl.BlockSpec((BKNN, 2), lambda i: (i, 0)),
            pl.BlockSpec((NCH, 2, CCH), lambda i: (0, 0, 0)),
        ],
        out_specs=[
            pl.BlockSpec((BKNN, K), lambda i: (i, 0)),
            pl.BlockSpec((BKNN, K), lambda i: (i, 0)),
            pl.BlockSpec((BKNN, 1), lambda i: (i, 0)),
        ],
        out_shape=[
            jax.ShapeDtypeStruct((NP, K), jnp.int32),
            jax.ShapeDtypeStruct((NP, K), jnp.float32),
            jax.ShapeDtypeStruct((NP, 1), jnp.float32),
        ],
        scratch_shapes=[pltpu.VMEM((NCH, BKNN, CCH), jnp.float32)],
    )(c_blocks, c_pad3)


def _sc_agg_body(ht_hbm, nbr_hbm, ew_hbm, out_hbm,
                 idx_v, ew_v, buf0, buf1, out_v, sem0, sem1):
    wid = lax.axis_index("s") * 2 + lax.axis_index("c")
    pltpu.sync_copy(nbr_hbm.at[wid], idx_v)
    pltpu.sync_copy(ew_hbm.at[wid], ew_v)

    def issue(cidx, buf, sem):
        pltpu.make_async_copy(ht_hbm.at[idx_v.at[cidx]], buf, sem).start()

    def wait(cidx, buf, sem):
        pltpu.make_async_copy(ht_hbm.at[idx_v.at[cidx]], buf, sem).wait()

    def compute(c, buf):
        def row_body(dloc, _):
            d = c * CHROWS + dloc
            acc = [jnp.zeros((16,), jnp.float32) for _ in range(8)]
            wv = [ew_v[d, pl.ds(0, 16)], ew_v[d, pl.ds(16, 16)]]
            for jn in range(K):
                w = wv[jn // 16][jn % 16]
                for t in range(8):
                    acc[t] = acc[t] + w * buf[dloc * K + jn, pl.ds(16 * t, 16)]
            for t in range(8):
                out_v[d, pl.ds(16 * t, 16)] = acc[t]
            return 0

        lax.fori_loop(0, CHROWS, row_body, 0, unroll=False)

    issue(0, buf0, sem0)

    def chunk_body(c, _):
        issue(c + 1, buf1, sem1)
        wait(c, buf0, sem0)
        compute(c, buf0)

        @pl.when(c + 2 < CPW)
        def _():
            issue(c + 2, buf0, sem0)

        wait(c + 1, buf1, sem1)
        compute(c + 1, buf1)
        return 0

    lax.fori_loop(0, CPW // 2, lambda i, carry: chunk_body(2 * i, carry), 0,
                  unroll=False)
    pltpu.sync_copy(out_v, out_hbm.at[pl.ds(wid * RPW, RPW)])


@functools.cache
def _sc_agg_call():
    mesh = plsc.VectorSubcoreMesh(core_axis_name="c", subcore_axis_name="s")
    return pl.kernel(
        _sc_agg_body,
        mesh=mesh,
        out_type=jax.ShapeDtypeStruct((NP, 128), jnp.float32),
        scratch_types=[
            pltpu.VMEM((CPW, GATHER), jnp.int32),
            pltpu.VMEM((RPW, K), jnp.float32),
            pltpu.VMEM((GATHER, 128), jnp.float32),
            pltpu.VMEM((GATHER, 128), jnp.float32),
            pltpu.VMEM((RPW, 128), jnp.float32),
            pltpu.SemaphoreType.DMA,
            pltpu.SemaphoreType.DMA,
        ],
    )


def _sc_agg(ht, nbr3, ew3):
    return _sc_agg_call()(ht, nbr3, ew3)


def _mm(a, w):
    return jnp.dot(a.astype(jnp.bfloat16), w.astype(jnp.bfloat16),
                   preferred_element_type=jnp.float32)


def _pre_body(c_ref, x_ref, dinv_ref, fcc_w, fcc_b, fca_w, fca_b, g1w,
              f1w, f1b, f2w, f2b, f3w, f3b,
              ht1_ref, coords0_ref, m3_ref):
    cb = c_ref[...]
    xb = x_ref[...]
    dinv = dinv_ref[...]
    coords0_ref[...] = _mm(cb, fcc_w[...]) + fcc_b[...]
    ht1_ref[...] = dinv * _mm(cb, g1w[...])
    attri0 = _mm(xb, fca_w[...]) + fca_b[...]
    m1 = attri0 + jax.nn.relu(_mm(xb, f1w[...]) + f1b[...])
    m2 = m1 + jax.nn.relu(_mm(m1, f2w[...]) + f2b[...])
    m3_ref[...] = m2 + jax.nn.relu(_mm(m2, f3w[...]) + f3b[...])


def _mid_body(s_ref, ht_ref, base_ref, dinv_ref, b_ref, w_next,
              g_ref, htn_ref):
    dinv = dinv_ref[...]
    conv = dinv * (s_ref[...] + ht_ref[...]) + b_ref[...]
    g = base_ref[...] + jax.nn.relu(conv)
    g_ref[...] = g
    htn_ref[...] = dinv * _mm(g, w_next[...])


def _head_body(s_ref, ht_ref, base_ref, m3_ref, dinv_ref, g3b,
               d1wa, d1wb, d1b, d2w, d2b,
               mean_ref, sigma_ref, g3_ref):
    dinv = dinv_ref[...]
    conv = dinv * (s_ref[...] + ht_ref[...]) + g3b[...]
    g3 = base_ref[...] + jax.nn.relu(conv)
    g3_ref[...] = g3
    h = jax.nn.relu(_mm(g3, d1wa[...]) + _mm(m3_ref[...], d1wb[...]) + d1b[...])
    out = _mm(h, d2w[...]) + d2b[...]
    mean_ref[...] = out[:, 0:1]
    std = out[:, 1:2]
    sp = jnp.log(1.0 + jnp.exp(-jnp.abs(std))) + jnp.maximum(std, 0.0)
    sigma_ref[...] = 0.2 + 0.8 * sp


def _row_spec(cols):
    return pl.BlockSpec((BD, cols), lambda i: (i, 0))


def _full_spec(r, cols):
    return pl.BlockSpec((r, cols), lambda i: (0, 0))


def _pre(c_pad, x_pad, dinv, fcc_w, fcc_b, fca_w, fca_b, g1w,
         f1w, f1b, f2w, f2b, f3w, f3b):
    grid = NP // BD
    return pl.pallas_call(
        _pre_body,
        grid=(grid,),
        in_specs=[
            _row_spec(2), _row_spec(128), _row_spec(1),
            _full_spec(2, 128), _full_spec(1, 128),
            _full_spec(128, 128), _full_spec(1, 128),
            _full_spec(2, 128),
            _full_spec(128, 128), _full_spec(1, 128),
            _full_spec(128, 128), _full_spec(1, 128),
            _full_spec(128, 128), _full_spec(1, 128),
        ],
        out_specs=[_row_spec(128), _row_spec(128), _row_spec(128)],
        out_shape=[jax.ShapeDtypeStruct((NP, 128), jnp.float32)] * 3,
    )(c_pad, x_pad, dinv, fcc_w, fcc_b, fca_w, fca_b, g1w,
      f1w, f1b, f2w, f2b, f3w, f3b)


def _mid(s, ht, base, dinv, b, w_next):
    grid = NP // BD
    return pl.pallas_call(
        _mid_body,
        grid=(grid,),
        in_specs=[
            _row_spec(128), _row_spec(128), _row_spec(128), _row_spec(1),
            _full_spec(1, 128), _full_spec(128, 128),
        ],
        out_specs=[_row_spec(128), _row_spec(128)],
        out_shape=[jax.ShapeDtypeStruct((NP, 128), jnp.float32)] * 2,
    )(s, ht, base, dinv, b, w_next)


def _head(s, ht, base, m3, dinv, g3b, d1wa, d1wb, d1b, d2w, d2b):
    grid = NP // BD
    return pl.pallas_call(
        _head_body,
        grid=(grid,),
        in_specs=[
            _row_spec(128), _row_spec(128), _row_spec(128), _row_spec(128),
            _row_spec(1),
            _full_spec(1, 128),
            _full_spec(128, 256), _full_spec(128, 256), _full_spec(1, 256),
            _full_spec(256, 2), _full_spec(1, 2),
        ],
        out_specs=[_row_spec(1), _row_spec(1), _row_spec(128)],
        out_shape=[
            jax.ShapeDtypeStruct((NP, 1), jnp.float32),
            jax.ShapeDtypeStruct((NP, 1), jnp.float32),
            jax.ShapeDtypeStruct((NP, 128), jnp.float32),
        ],
    )(s, ht, base, m3, dinv, g3b, d1wa, d1wb, d1b, d2w, d2b)


def kernel(c, x, fc_c_W, fc_c_b, fc_a_W, fc_a_b, g1_W, g1_b, g2_W, g2_b,
           g3_W, g3_b, f1_W, f1_b, f2_W, f2_b, f3_W, f3_b,
           d1_W, d1_b, d2_W, d2_b):
    pad = NP - N
    c_pad = jnp.concatenate([c, jnp.full((pad, 2), 1e9, jnp.float32)], axis=0)
    x_pad = jnp.concatenate([x, jnp.zeros((pad, 128), jnp.float32)], axis=0)
    c_pad3 = c_pad.T.reshape(2, NCH, CCH).transpose(1, 0, 2)

    nbr, ew, dinv = _knn(c_pad, c_pad3)
    nbr3 = nbr.reshape(NW, CPW, GATHER)
    ew3 = ew.reshape(NW, RPW, K)

    row = lambda b: b.reshape(1, -1)

    ht1, coords0, m3 = _pre(
        c_pad, x_pad, dinv, fc_c_W, row(fc_c_b), fc_a_W, row(fc_a_b), g1_W,
        f1_W, row(f1_b), f2_W, row(f2_b), f3_W, row(f3_b))

    s1 = _sc_agg(ht1, nbr3, ew3)
    g1, ht2 = _mid(s1, ht1, coords0, dinv, row(g1_b), g2_W)

    s2 = _sc_agg(ht2, nbr3, ew3)
    g2, ht3 = _mid(s2, ht2, g1, dinv, row(g2_b), g3_W)

    s3 = _sc_agg(ht3, nbr3, ew3)
    mean, sigma, g3 = _head(
        s3, ht3, g2, m3, dinv, row(g3_b),
        d1_W[:128], d1_W[128:], row(d1_b), d2_W, row(d2_b))

    return (mean[:N], sigma[:N], g3[:N], m3[:N])


# speedup vs baseline: 3.5151x; 1.0378x over previous
import functools

import jax
import jax.numpy as jnp
from jax import lax
from jax.experimental import pallas as pl
from jax.experimental.pallas import tpu as pltpu
from jax.experimental.pallas import tpu_sc as plsc

N = 10000
K = 32
NP = 10240
BKNN = 128
BD = 1024
NW = 32
RPW = NP // NW
CHROWS = 4
GATHER = CHROWS * K
CPW = RPW // CHROWS

_HIGH = jax.lax.Precision.HIGHEST


CCH = 1024
NCH = NP // CCH


def _knn_body(cb_ref, ct_ref, idx_ref, ew_ref, dinv_ref, d2_ref):
    cb = cb_ref[...]
    cx = cb[:, 0:1]
    cy = cb[:, 1:2]
    r = pl.program_id(0) * BKNN + lax.broadcasted_iota(jnp.int32, (BKNN, 1), 0)
    jj = lax.broadcasted_iota(jnp.int32, (BKNN, CCH), 1)

    def fill(cc, _):
        ctx = ct_ref[cc, 0:1, :]
        cty = ct_ref[cc, 1:2, :]
        dx = cx - ctx
        dy = cy - cty
        d2 = dx * dx + dy * dy
        col = jj + cc * CCH
        d2_ref[cc] = jnp.where(col == r, jnp.inf, d2)
        return 0

    lax.fori_loop(0, NCH, fill, 0, unroll=False)

    lanes = lax.broadcasted_iota(jnp.int32, (BKNN, NCH), 1)

    def cache0(cc, carry):
        cm, ca = carry
        x = d2_ref[cc]
        col = jj + cc * CCH
        nm = jnp.min(x, axis=1, keepdims=True)
        na = jnp.min(jnp.where(x == nm, col, NP), axis=1, keepdims=True)
        cm = jnp.where(lanes == cc, nm, cm)
        ca = jnp.where(lanes == cc, na, ca)
        return cm, ca

    cm0 = jnp.full((BKNN, NCH), jnp.inf, jnp.float32)
    ca0 = jnp.full((BKNN, NCH), NP, jnp.int32)
    cm, ca = lax.fori_loop(0, NCH, cache0, (cm0, ca0), unroll=False)

    def select(k, carry):
        vals, idxs, cm, ca = carry
        m = jnp.min(cm, axis=1, keepdims=True)
        i1 = jnp.min(jnp.where(cm == m, ca, NP), axis=1, keepdims=True)
        kcol = lax.broadcasted_iota(jnp.int32, (BKNN, K), 1)
        vals = jnp.where(kcol == k, m, vals)
        idxs = jnp.where(kcol == k, i1, idxs)

        def upd(cc, carry2):
            cmi, cai = carry2
            col = jj + cc * CCH
            msk = jnp.where(col == i1, jnp.inf, d2_ref[cc])
            d2_ref[cc] = msk
            nm = jnp.min(msk, axis=1, keepdims=True)
            na = jnp.min(jnp.where(msk == nm, col, NP), axis=1, keepdims=True)
            cmi = jnp.where(lanes == cc, nm, cmi)
            cai = jnp.where(lanes == cc, na, cai)
            return cmi, cai

        cm, ca = lax.fori_loop(0, NCH, upd, (cm, ca), unroll=False)
        return vals, idxs, cm, ca

    vals0 = jnp.zeros((BKNN, K), jnp.float32)
    idxs0 = jnp.zeros((BKNN, K), jnp.int32)
    v, ii, cm, ca = lax.fori_loop(0, K, select, (vals0, idxs0, cm, ca),
                                  unroll=False)
    ew = jnp.exp(-jnp.sqrt(v) / 2.0)
    idx_ref[...] = ii
    ew_ref[...] = ew
    dinv_ref[...] = 1.0 / jnp.sqrt(1.0 + jnp.sum(ew, axis=1, keepdims=True))


def _knn(c_blocks, c_pad3):
    grid = NP // BKNN
    return pl.pallas_call(
        _knn_body,
        grid=(grid,),
        in_specs=[
            pl.BlockSpec((BKNN, 2), lambda i: (i, 0)),
            pl.BlockSpec((NCH, 2, CCH), lambda i: (0, 0, 0)),
        ],
        out_specs=[
            pl.BlockSpec((BKNN, K), lambda i: (i, 0)),
            pl.BlockSpec((BKNN, K), lambda i: (i, 0)),
            pl.BlockSpec((BKNN, 1), lambda i: (i, 0)),
        ],
        out_shape=[
            jax.ShapeDtypeStruct((NP, K), jnp.int32),
            jax.ShapeDtypeStruct((NP, K), jnp.float32),
            jax.ShapeDtypeStruct((NP, 1), jnp.float32),
        ],
        scratch_shapes=[pltpu.VMEM((NCH, BKNN, CCH), jnp.float32)],
    )(c_blocks, c_pad3)


def _sc_agg_body(ht_hbm, nbr_hbm, ew_hbm, out_hbm,
                 idx_v, ew_v, buf0, buf1, out_v, sem0, sem1):
    wid = lax.axis_index("s") * 2 + lax.axis_index("c")
    pltpu.sync_copy(nbr_hbm.at[wid], idx_v)
    pltpu.sync_copy(ew_hbm.at[wid], ew_v)

    def issue(cidx, buf, sem):
        pltpu.make_async_copy(ht_hbm.at[idx_v.at[cidx]], buf, sem).start()

    def wait(cidx, buf, sem):
        pltpu.make_async_copy(ht_hbm.at[idx_v.at[cidx]], buf, sem).wait()

    def compute(c, buf):
        def row_body(dloc, _):
            d = c * CHROWS + dloc
            acc = [jnp.zeros((16,), jnp.float32) for _ in range(8)]
            wv = [ew_v[d, pl.ds(0, 16)], ew_v[d, pl.ds(16, 16)]]
            for jn in range(K):
                w = wv[jn // 16][jn % 16]
                for t in range(8):
                    acc[t] = acc[t] + w * buf[dloc * K + jn, pl.ds(16 * t, 16)]
            for t in range(8):
                out_v[d, pl.ds(16 * t, 16)] = acc[t]
            return 0

        lax.fori_loop(0, CHROWS, row_body, 0, unroll=False)

    issue(0, buf0, sem0)

    def chunk_body(c, _):
        issue(c + 1, buf1, sem1)
        wait(c, buf0, sem0)
        compute(c, buf0)

        @pl.when(c + 2 < CPW)
        def _():
            issue(c + 2, buf0, sem0)

        wait(c + 1, buf1, sem1)
        compute(c + 1, buf1)
        return 0

    lax.fori_loop(0, CPW // 2, lambda i, carry: chunk_body(2 * i, carry), 0,
                  unroll=False)
    pltpu.sync_copy(out_v, out_hbm.at[pl.ds(wid * RPW, RPW)])


@functools.cache
def _sc_agg_call():
    mesh = plsc.VectorSubcoreMesh(core_axis_name="c", subcore_axis_name="s")
    return pl.kernel(
        _sc_agg_body,
        mesh=mesh,
        out_type=jax.ShapeDtypeStruct((NP, 128), jnp.float32),
        scratch_types=[
            pltpu.VMEM((CPW, GATHER), jnp.int32),
            pltpu.VMEM((RPW, K), jnp.float32),
            pltpu.VMEM((GATHER, 128), jnp.float32),
            pltpu.VMEM((GATHER, 128), jnp.float32),
            pltpu.VMEM((RPW, 128), jnp.float32),
            pltpu.SemaphoreType.DMA,
            pltpu.SemaphoreType.DMA,
        ],
    )


def _sc_agg(ht, nbr3, ew3):
    return _sc_agg_call()(ht, nbr3, ew3)


def _mm(a, w):
    return jnp.dot(a.astype(jnp.bfloat16), w.astype(jnp.bfloat16),
                   preferred_element_type=jnp.float32)


def _pre_body(c_ref, x_ref, dinv_ref, fcc_w, fcc_b, fca_w, fca_b, g1w,
              f1w, f1b, f2w, f2b, f3w, f3b,
              ht1_ref, coords0_ref, m3_ref):
    cb = c_ref[...]
    xb = x_ref[...]
    dinv = dinv_ref[...]
    coords0_ref[...] = _mm(cb, fcc_w[...]) + fcc_b[...]
    ht1_ref[...] = dinv * _mm(cb, g1w[...])
    attri0 = _mm(xb, fca_w[...]) + fca_b[...]
    m1 = attri0 + jax.nn.relu(_mm(xb, f1w[...]) + f1b[...])
    m2 = m1 + jax.nn.relu(_mm(m1, f2w[...]) + f2b[...])
    m3_ref[...] = m2 + jax.nn.relu(_mm(m2, f3w[...]) + f3b[...])


def _mid_body(s_ref, ht_ref, base_ref, dinv_ref, b_ref, w_next,
              g_ref, htn_ref):
    dinv = dinv_ref[...]
    conv = dinv * (s_ref[...] + ht_ref[...]) + b_ref[...]
    g = base_ref[...] + jax.nn.relu(conv)
    g_ref[...] = g
    htn_ref[...] = dinv * _mm(g, w_next[...])


def _head_body(s_ref, ht_ref, base_ref, m3_ref, dinv_ref, g3b,
               d1wa, d1wb, d1b, d2w, d2b,
               mean_ref, sigma_ref, g3_ref):
    dinv = dinv_ref[...]
    conv = dinv * (s_ref[...] + ht_ref[...]) + g3b[...]
    g3 = base_ref[...] + jax.nn.relu(conv)
    g3_ref[...] = g3
    h = jax.nn.relu(_mm(g3, d1wa[...]) + _mm(m3_ref[...], d1wb[...]) + d1b[...])
    out = _mm(h, d2w[...]) + d2b[...]
    mean_ref[...] = out[:, 0:1]
    std = out[:, 1:2]
    sp = jnp.log(1.0 + jnp.exp(-jnp.abs(std))) + jnp.maximum(std, 0.0)
    sigma_ref[...] = 0.2 + 0.8 * sp


def _row_spec(cols):
    return pl.BlockSpec((BD, cols), lambda i: (i, 0))


def _full_spec(r, cols):
    return pl.BlockSpec((r, cols), lambda i: (0, 0))


def _pre(c_pad, x_pad, dinv, fcc_w, fcc_b, fca_w, fca_b, g1w,
         f1w, f1b, f2w, f2b, f3w, f3b):
    grid = NP // BD
    return pl.pallas_call(
        _pre_body,
        grid=(grid,),
        in_specs=[
            _row_spec(2), _row_spec(128), _row_spec(1),
            _full_spec(2, 128), _full_spec(1, 128),
            _full_spec(128, 128), _full_spec(1, 128),
            _full_spec(2, 128),
            _full_spec(128, 128), _full_spec(1, 128),
            _full_spec(128, 128), _full_spec(1, 128),
            _full_spec(128, 128), _full_spec(1, 128),
        ],
        out_specs=[_row_spec(128), _row_spec(128), _row_spec(128)],
        out_shape=[jax.ShapeDtypeStruct((NP, 128), jnp.float32)] * 3,
    )(c_pad, x_pad, dinv, fcc_w, fcc_b, fca_w, fca_b, g1w,
      f1w, f1b, f2w, f2b, f3w, f3b)


def _mid(s, ht, base, dinv, b, w_next):
    grid = NP // BD
    return pl.pallas_call(
        _mid_body,
        grid=(grid,),
        in_specs=[
            _row_spec(128), _row_spec(128), _row_spec(128), _row_spec(1),
            _full_spec(1, 128), _full_spec(128, 128),
        ],
        out_specs=[_row_spec(128), _row_spec(128)],
        out_shape=[jax.ShapeDtypeStruct((NP, 128), jnp.float32)] * 2,
    )(s, ht, base, dinv, b, w_next)


def _head(s, ht, base, m3, dinv, g3b, d1wa, d1wb, d1b, d2w, d2b):
    grid = NP // BD
    return pl.pallas_call(
        _head_body,
        grid=(grid,),
        in_specs=[
            _row_spec(128), _row_spec(128), _row_spec(128), _row_spec(128),
            _row_spec(1),
            _full_spec(1, 128),
            _full_spec(128, 256), _full_spec(128, 256), _full_spec(1, 256),
            _full_spec(256, 2), _full_spec(1, 2),
        ],
        out_specs=[_row_spec(1), _row_spec(1), _row_spec(128)],
        out_shape=[
            jax.ShapeDtypeStruct((NP, 1), jnp.float32),
            jax.ShapeDtypeStruct((NP, 1), jnp.float32),
            jax.ShapeDtypeStruct((NP, 128), jnp.float32),
        ],
    )(s, ht, base, m3, dinv, g3b, d1wa, d1wb, d1b, d2w, d2b)


def kernel(c, x, fc_c_W, fc_c_b, fc_a_W, fc_a_b, g1_W, g1_b, g2_W, g2_b,
           g3_W, g3_b, f1_W, f1_b, f2_W, f2_b, f3_W, f3_b,
           d1_W, d1_b, d2_W, d2_b):
    pad = NP - N
    c_pad = jnp.concatenate([c, jnp.full((pad, 2), 1e9, jnp.float32)], axis=0)
    x_pad = jnp.concatenate([x, jnp.zeros((pad, 128), jnp.float32)], axis=0)
    c_pad3 = c_pad.T.reshape(2, NCH, CCH).transpose(1, 0, 2)

    nbr, ew, dinv = _knn(c_pad, c_pad3)
    nbr3 = nbr.reshape(NW, CPW, GATHER)
    ew3 = ew.reshape(NW, RPW, K)

    row = lambda b: b.reshape(1, -1)

    ht1, coords0, m3 = _pre(
        c_pad, x_pad, dinv, fc_c_W, row(fc_c_b), fc_a_W, row(fc_a_b), g1_W,
        f1_W, row(f1_b), f2_W, row(f2_b), f3_W, row(f3_b))

    s1 = _sc_agg(ht1, nbr3, ew3)
    g1, ht2 = _mid(s1, ht1, coords0, dinv, row(g1_b), g2_W)

    s2 = _sc_agg(ht2, nbr3, ew3)
    g2, ht3 = _mid(s2, ht2, g1, dinv, row(g2_b), g3_W)

    s3 = _sc_agg(ht3, nbr3, ew3)
    mean, sigma, g3 = _head(
        s3, ht3, g2, m3, dinv, row(g3_b),
        d1_W[:128], d1_W[128:], row(d1_b), d2_W, row(d2_b))

    return (mean[:N], sigma[:N], g3[:N], m3[:N])
